# baseline (speedup 1.0000x reference)
import functools
import math

import jax
import jax.numpy as jnp
import numpy as np
from jax import lax
from jax.experimental import pallas as pl
from jax.experimental.pallas import tpu as pltpu
from jax.experimental.pallas import tpu_sc as plsc

_NC = 80
_NO = _NC + 5
_NP = 3072
_BAL = (4.0, 1.0, 0.4)
_H_GIOU, _H_OBJ, _H_CLS = 0.05, 1.0, 0.5
_EPS = 1e-9


def _build_targets(pshapes, targets, anchors, anchor_t):
    na, nt = anchors.shape[1], targets.shape[0]
    tcls, tbox, rows_l, anch, masks = [], [], [], [], []
    ai = jnp.tile(jnp.arange(na, dtype=jnp.float32).reshape(na, 1), (1, nt))
    t_all = jnp.concatenate((jnp.tile(targets[None], (na, 1, 1)), ai[:, :, None]), axis=2)
    g = 0.5
    off = jnp.array([[0, 0], [1, 0], [0, 1], [-1, 0], [0, -1]], dtype=jnp.float32) * g
    anchor_t_f = jnp.asarray(anchor_t, dtype=jnp.float32)
    for i in range(len(pshapes)):
        B, _, H, W, _ = pshapes[i]
        anc = anchors[i]
        gain = np.ones(7, dtype=np.float32)
        gain[2:6] = np.array([W, H, W, H], dtype=np.float32)
        gain_j = jnp.asarray(gain)
        t = t_all * gain_j
        r = t[:, :, 4:6] / anc[:, None, :]
        jmask0 = jnp.max(jnp.maximum(r, 1.0 / r), axis=2) < anchor_t_f
        tf = t.reshape(na * nt, 7)
        m0 = jmask0.reshape(na * nt)
        gxy = tf[:, 2:4]
        gxi = gain_j[2:4] - gxy
        jk = (gxy % 1.0 < g) & (gxy > 1.0)
        lm = (gxi % 1.0 < g) & (gxi > 1.0)
        jmask = jnp.stack((jnp.ones(na * nt, dtype=bool), jk[:, 0], jk[:, 1], lm[:, 0], lm[:, 1])) & m0[None]
        tt = jnp.broadcast_to(tf[None], (5, na * nt, 7)).reshape(5 * na * nt, 7)
        offsets = jnp.broadcast_to(off[:, None, :], (5, na * nt, 2)).reshape(5 * na * nt, 2)
        m = jmask.reshape(5 * na * nt)
        b = tt[:, 0].astype(jnp.int32)
        c = tt[:, 1]
        gxy2 = tt[:, 2:4]
        gwh = tt[:, 4:6]
        gij = (gxy2 - offsets).astype(jnp.int32)
        gi = jnp.clip(gij[:, 0], 0, W - 1)
        gj = jnp.clip(gij[:, 1], 0, H - 1)
        a = tt[:, 6].astype(jnp.int32)
        rows = ((b * na + a) * H + gj) * W + gi
        rows_l.append(rows)
        tbox.append(jnp.concatenate(
            (gxy2 - jnp.stack([gi, gj], axis=1).astype(jnp.float32), gwh), axis=1))
        anch.append(anc[a])
        tcls.append(c)
        masks.append(m)
    return tcls, tbox, rows_l, anch, masks


def _softplus(x):
    return jnp.maximum(x, 0.0) + jnp.log(1.0 + jnp.exp(-jnp.abs(x)))


def _sigmoid(x):
    return 1.0 / (1.0 + jnp.exp(-x))


def _atan_pos(x):
    big = x > 2.414213562373095
    mid = x > 0.4142135623730951
    xr = jnp.where(big, -1.0 / jnp.maximum(x, 1e-30),
                   jnp.where(mid, (x - 1.0) / (x + 1.0), x))
    z = xr * xr
    y = ((((8.05374449538e-2 * z - 1.38776856032e-1) * z + 1.99777106478e-1) * z
          - 3.33329491539e-1) * z) * xr + xr
    return jnp.where(big, math.pi / 2 + y, jnp.where(mid, math.pi / 4 + y, y))


def _sc_call(p0f, p1f, p2f, rows2d, scat2d):
    cells = [p0f.shape[0], p1f.shape[0], p2f.shape[0]]
    mesh = plsc.VectorSubcoreMesh(core_axis_name="c", subcore_axis_name="s",
                                  num_cores=2)
    out_type = [
        jax.ShapeDtypeStruct((3 * _NP, _NO), jnp.float32),
        jax.ShapeDtypeStruct((3 * _NP,), jnp.int32),
        jax.ShapeDtypeStruct((cells[0] + 16,), jnp.int32),
        jax.ShapeDtypeStruct((cells[1] + 16,), jnp.int32),
        jax.ShapeDtypeStruct((cells[2] + 16,), jnp.int32),
    ]
    scratch = [
        pltpu.VMEM((96,), jnp.int32),
        pltpu.VMEM((96,), jnp.int32),
        pltpu.VMEM((96,), jnp.int32),
        pltpu.VMEM((96,), jnp.int32),
        pltpu.VMEM((96, _NO), jnp.float32),
        pltpu.SemaphoreType.DMA,
    ]

    @functools.partial(pl.kernel, mesh=mesh, out_type=out_type,
                       scratch_types=scratch)
    def k(p0h, p1h, p2h, rowsh, scath, ps_out, g_out, m0, m1, m2,
          idx0, idx1, valv, gv, psv, sem):
        c = lax.axis_index("c")
        s = lax.axis_index("s")
        maps = [m0, m1, m2]
        phs = [p0h, p1h, p2h]

        @pl.when(c == 1)
        def _gather_ps():
            for lvl in range(3):
                for ch, idxv in ((0, idx0), (1, idx1)):
                    base = s * 192 + ch * 96
                    pltpu.sync_copy(rowsh.at[pl.ds(lvl * _NP + base, 96)], idxv)
                    pltpu.async_copy(phs[lvl].at[idxv], psv, sem).wait()
                    pltpu.sync_copy(psv, ps_out.at[pl.ds(lvl * _NP + base, 96), :])

        @pl.when(c == 0)
        def _scatter_ord():
            for lvl in range(3):
                for ch, idxv in ((0, idx0), (1, idx1)):
                    base = s * 192 + ch * 96
                    pltpu.sync_copy(scath.at[pl.ds(lvl * _NP + base, 96)], idxv)
                    for t in range(6):
                        valv[pl.ds(t * 16, 16)] = (
                            lax.iota(jnp.int32, 16) + (lvl * _NP + base + t * 16))
                    pltpu.async_copy(valv, maps[lvl].at[idxv], sem).wait()

        plsc.subcore_barrier()

        @pl.when(c == 0)
        def _gather_ord():
            for lvl in range(3):
                for ch, idxv in ((0, idx0), (1, idx1)):
                    base = s * 192 + ch * 96
                    pltpu.sync_copy(scath.at[pl.ds(lvl * _NP + base, 96)], idxv)
                    pltpu.async_copy(maps[lvl].at[idxv], gv, sem).wait()
                    pltpu.sync_copy(gv, g_out.at[pl.ds(lvl * _NP + base, 96)])

    return k(p0f, p1f, p2f, rows2d, scat2d)


def _entry_kernel(ps_ref, aux_ref, g_ref, gr_ref, sums_ref):
    ps = ps_ref[...]
    tbx = aux_ref[:, 0:1]
    tby = aux_ref[:, 1:2]
    tbw = aux_ref[:, 2:3]
    tbh = aux_ref[:, 3:4]
    anw = aux_ref[:, 4:5]
    anh = aux_ref[:, 5:6]
    mf = aux_ref[:, 6:7]
    tcl = aux_ref[:, 7:8]
    gr = gr_ref[0]

    px = _sigmoid(ps[:, 0:1]) * 2.0 - 0.5
    py = _sigmoid(ps[:, 1:2]) * 2.0 - 0.5
    pw = (_sigmoid(ps[:, 2:3]) * 2.0) ** 2 * anw
    ph = (_sigmoid(ps[:, 3:4]) * 2.0) ** 2 * anh

    b1x1, b1x2 = px - pw * 0.5, px + pw * 0.5
    b1y1, b1y2 = py - ph * 0.5, py + ph * 0.5
    b2x1, b2x2 = tbx - tbw * 0.5, tbx + tbw * 0.5
    b2y1, b2y2 = tby - tbh * 0.5, tby + tbh * 0.5
    inter = jnp.clip(jnp.minimum(b1x2, b2x2) - jnp.maximum(b1x1, b2x1), 0.0, None) * \
            jnp.clip(jnp.minimum(b1y2, b2y2) - jnp.maximum(b1y1, b2y1), 0.0, None)
    union = pw * ph + tbw * tbh - inter + _EPS
    iou = inter / union
    cw = jnp.maximum(b1x2, b2x2) - jnp.minimum(b1x1, b2x1)
    ch = jnp.maximum(b1y2, b2y2) - jnp.minimum(b1y1, b2y1)
    c2 = cw ** 2 + ch ** 2 + _EPS
    rho2 = ((b2x1 + b2x2 - b1x1 - b1x2) ** 2 + (b2y1 + b2y2 - b1y1 - b1y2) ** 2) / 4.0
    v = (4.0 / math.pi ** 2) * (_atan_pos(tbw / (tbh + _EPS)) - _atan_pos(pw / (ph + _EPS))) ** 2
    alpha = v / (1.0 - iou + v + _EPS)
    giou = iou - (rho2 / c2 + v * alpha)

    lbox_sum = jnp.sum(mf * (1.0 - giou))
    objt = (1.0 - gr) + gr * jnp.clip(giou, 0.0, None)

    winner = (g_ref[...] > 0.0) & (mf > 0.0)
    corr_sum = jnp.sum(jnp.where(winner, objt * ps[:, 4:5], 0.0))

    xc = ps[:, 5:_NO]
    lane = jax.lax.broadcasted_iota(jnp.int32, (xc.shape[0], _NC), 1)
    x_true = jnp.sum(jnp.where(lane == tcl.astype(jnp.int32), xc, 0.0), axis=1, keepdims=True)
    row_elem = jnp.sum(_softplus(xc), axis=1, keepdims=True) - x_true
    lcls_sum = jnp.sum(mf * row_elem)
    cnt = jnp.sum(mf)

    sums_ref[0, 0, 0] = lbox_sum
    sums_ref[0, 0, 1] = lcls_sum
    sums_ref[0, 0, 2] = cnt
    sums_ref[0, 0, 3] = corr_sum


def _obj_sp_kernel(p_ref, out_ref):
    x = p_ref[:, 4:5]
    partial = jnp.sum(_softplus(x))

    @pl.when(pl.program_id(0) == 0)
    def _init():
        out_ref[0] = 0.0

    out_ref[0] += partial


def _pad(x, n, axis=0):
    pads = [(0, 0)] * x.ndim
    pads[axis] = (0, n - x.shape[axis])
    return jnp.pad(x, pads)


def kernel(p0, p1, p2, targets, anchors, anchor_t, gr):
    preds = [p0, p1, p2]
    pshapes = [p.shape for p in preds]
    na = anchors.shape[1]
    tcls, tbox, rows_l, anch, masks = _build_targets(pshapes, targets, anchors, anchor_t)

    gr_f = jnp.asarray(gr, dtype=jnp.float32).reshape(1)

    rows2d, scat2d, aux_levels, cells_l = [], [], [], []
    for i in range(3):
        B, _, H, W, _ = pshapes[i]
        cells = B * na * H * W
        cells_l.append(cells)
        rows2d.append(_pad(rows_l[i], _NP))
        scat2d.append(jnp.where(_pad(masks[i], _NP), _pad(rows_l[i], _NP), cells))
        aux = jnp.concatenate([
            tbox[i], anch[i],
            masks[i].astype(jnp.float32)[:, None],
            tcls[i][:, None],
        ], axis=1)
        aux_levels.append(_pad(aux, _NP))
    aux_all = jnp.concatenate(aux_levels, axis=0)

    ps_levels, win_levels = [], []
    for i, pi in enumerate(preds):
        flat = pi.reshape(cells_l[i], _NO)
        ps_levels.append(flat[rows2d[i]])
        key = scat2d[i]
        perm = jnp.argsort(key, stable=True)
        key_s = key[perm]
        last = jnp.concatenate([key_s[1:] != key_s[:-1], jnp.ones((1,), bool)])
        win = jnp.zeros((_NP,), jnp.float32).at[perm].set(last.astype(jnp.float32))
        win_levels.append(win)
    ps_all = jnp.concatenate(ps_levels, axis=0)
    win_all = jnp.concatenate(win_levels)

    sums = pl.pallas_call(
        _entry_kernel,
        grid=(3,),
        in_specs=[
            pl.BlockSpec((_NP, _NO), lambda i: (i, 0)),
            pl.BlockSpec((_NP, 8), lambda i: (i, 0)),
            pl.BlockSpec((_NP, 1), lambda i: (i, 0)),
            pl.BlockSpec(memory_space=pltpu.SMEM),
        ],
        out_specs=pl.BlockSpec((1, 1, 4), lambda i: (i, 0, 0), memory_space=pltpu.SMEM),
        out_shape=jax.ShapeDtypeStruct((3, 1, 4), jnp.float32),
    )(ps_all, aux_all, win_all.reshape(3 * _NP, 1), gr_f)

    lbox = jnp.zeros((1,), jnp.float32)
    lcls = jnp.zeros((1,), jnp.float32)
    lobj = jnp.zeros((1,), jnp.float32)
    for i, pi in enumerate(preds):
        cells = cells_l[i]
        rb = {0: 16384, 1: 16384, 2: 12288}[i]
        acc = pl.pallas_call(
            _obj_sp_kernel,
            grid=(cells // rb,),
            in_specs=[pl.BlockSpec((rb, _NO), lambda k: (k, 0))],
            out_specs=pl.BlockSpec(memory_space=pltpu.SMEM),
            out_shape=jax.ShapeDtypeStruct((1,), jnp.float32),
        )(pi.reshape(cells, _NO))

        cnt = sums[i, 0, 2]
        lbox += jnp.where(cnt > 0, sums[i, 0, 0] / cnt, 0.0)
        lcls += jnp.where(cnt > 0, sums[i, 0, 1] / (cnt * _NC), 0.0)
        lobj += (acc - sums[i, 0, 3]) * (_BAL[i] / cells)

    s = 3.0 / len(preds)
    lbox = lbox * _H_GIOU * s
    lobj = lobj * _H_OBJ * s
    lcls = lcls * _H_CLS * s
    bs = preds[-1].shape[0]
    loss = lbox + lobj + lcls
    return (loss * bs, jax.lax.stop_gradient(jnp.concatenate((lbox, lobj, lcls, loss))))
